# baseline (speedup 1.0000x reference)
import functools

import jax
import jax.numpy as jnp
from jax import lax
from jax.experimental import pallas as pl
from jax.experimental.pallas import tpu as pltpu
from jax.experimental.pallas import tpu_sc as plsc

NUM_ROWS = 16384
NUM_COLS = 26
EMB = 64
TOTAL = NUM_ROWS * NUM_COLS
NW = 32
PER_W = TOTAL // NW
GRP = 128
NGRP = PER_W // GRP


def _make_kernel():
    mesh = plsc.VectorSubcoreMesh(core_axis_name="c", subcore_axis_name="s")

    @functools.partial(
        pl.kernel,
        mesh=mesh,
        out_type=jax.ShapeDtypeStruct((NW, PER_W, EMB), jnp.float32),
        scratch_types=[
            pltpu.VMEM((NGRP, GRP), jnp.int32),
            pltpu.VMEM((GRP, EMB), jnp.float32),
            pltpu.SemaphoreType.DMA,
        ],
        compiler_params=pltpu.CompilerParams(use_tc_tiling_on_sc=False),
    )
    def gather_kernel(idx_hbm, table_hbm, out_hbm, idx_v, rows_v, sem):
        cid = lax.axis_index("c")
        sid = lax.axis_index("s")
        wid = sid * 2 + cid
        pltpu.sync_copy(idx_hbm.at[wid], idx_v)

        def body(j, carry):
            pltpu.async_copy(table_hbm.at[idx_v.at[j]], rows_v, sem).wait()
            pltpu.sync_copy(rows_v, out_hbm.at[wid].at[pl.ds(j * GRP, GRP)])
            return carry

        lax.fori_loop(0, NGRP, body, 0)

    return gather_kernel


_gather = _make_kernel()


def kernel(x, logits):
    idx = x.reshape(NW, NGRP, GRP)
    out = _gather(idx, logits)
    return out.reshape(NUM_ROWS, NUM_COLS, EMB)
